# baseline (speedup 1.0000x reference)
import functools
import jax
import jax.numpy as jnp
from jax import lax
from jax.experimental import pallas as pl
from jax.experimental.pallas import tpu as pltpu
from jax.experimental.pallas import tpu_sc as plsc

D_MODEL = 768
D_FF = 2048
NE = 8
TOPK = 2
T = 2048
NP = T * TOPK
BT = 256
N_TILES = NP // BT + NE
N_MAX = N_TILES * BT
NW = 32
GPW = N_MAX // NW
GCH = 64
TPW = T // NW
CCH = 32


def _ffn_tile(te_ref, nl_ref, xs_ref, g_ref, w1_ref, w3_ref, w2_ref, out_ref):
    i = pl.program_id(0)

    @pl.when(i < nl_ref[0])
    def _():
        xv = xs_ref[...].astype(jnp.bfloat16)
        w1 = w1_ref[0].astype(jnp.bfloat16)
        w3 = w3_ref[0].astype(jnp.bfloat16)
        h = jnp.dot(xv, w1, preferred_element_type=jnp.float32)
        g = jnp.dot(xv, w3, preferred_element_type=jnp.float32)
        a = ((h * jax.nn.sigmoid(h)) * g).astype(jnp.bfloat16)
        w2 = w2_ref[0].astype(jnp.bfloat16)
        y = jnp.dot(a, w2, preferred_element_type=jnp.float32)
        out_ref[...] = y * g_ref[0, 0][:, None]


def _ffn(te, nl, xs, gate_tiles, W1, W3, W2):
    grid_spec = pltpu.PrefetchScalarGridSpec(
        num_scalar_prefetch=2,
        grid=(N_TILES,),
        in_specs=[
            pl.BlockSpec((BT, D_MODEL), lambda i, te, nl: (i, 0)),
            pl.BlockSpec((1, 1, BT), lambda i, te, nl: (i, 0, 0)),
            pl.BlockSpec((1, D_MODEL, D_FF), lambda i, te, nl: (te[i], 0, 0)),
            pl.BlockSpec((1, D_MODEL, D_FF), lambda i, te, nl: (te[i], 0, 0)),
            pl.BlockSpec((1, D_FF, D_MODEL), lambda i, te, nl: (te[i], 0, 0)),
        ],
        out_specs=pl.BlockSpec((BT, D_MODEL), lambda i, te, nl: (i, 0)),
    )
    return pl.pallas_call(
        _ffn_tile,
        grid_spec=grid_spec,
        out_shape=jax.ShapeDtypeStruct((N_MAX, D_MODEL), jnp.float32),
        compiler_params=pltpu.CompilerParams(
            dimension_semantics=("arbitrary",)),
    )(te, nl, xs, gate_tiles, W1, W3, W2)


@functools.cache
def _sc_mesh():
    return plsc.VectorSubcoreMesh(
        core_axis_name="c", subcore_axis_name="s", num_cores=2)


def _gather_rows_body(x_hbm, idx_hbm, out_hbm, idx_v, rows_a, rows_b, sem_a, sem_b):
    wid = lax.axis_index("s") * 2 + lax.axis_index("c")
    base = wid * GPW
    pltpu.sync_copy(idx_hbm.at[pl.ds(base, GPW)], idx_v)
    bufs = (rows_a, rows_b)
    sems = (sem_a, sem_b)
    nch = GPW // GCH
    cps = [None, None]
    cps[0] = pltpu.async_copy(
        x_hbm.at[idx_v.at[pl.ds(0, GCH)]], rows_a, sem_a)
    for c in range(nch):
        if c + 1 < nch:
            cps[(c + 1) % 2] = pltpu.async_copy(
                x_hbm.at[idx_v.at[pl.ds((c + 1) * GCH, GCH)]],
                bufs[(c + 1) % 2], sems[(c + 1) % 2])
        cps[c % 2].wait()
        pltpu.sync_copy(bufs[c % 2], out_hbm.at[pl.ds(base + c * GCH, GCH)])


@functools.cache
def _gather_rows():
    return functools.partial(
        pl.kernel, mesh=_sc_mesh(),
        out_type=jax.ShapeDtypeStruct((N_MAX, D_MODEL), jnp.float32),
        scratch_types=[
            pltpu.VMEM((GPW,), jnp.int32),
            pltpu.VMEM((GCH, D_MODEL), jnp.float32),
            pltpu.VMEM((GCH, D_MODEL), jnp.float32),
            pltpu.SemaphoreType.DMA,
            pltpu.SemaphoreType.DMA,
        ],
    )(_gather_rows_body)


def _combine_rows_body(ys_hbm, p0_hbm, p1_hbm, out_hbm, i0_v, i1_v, r0_v, r1_v, s0, s1):
    wid = lax.axis_index("s") * 2 + lax.axis_index("c")
    base = wid * TPW

    def chunk(c, carry):
        off = base + c * CCH
        pltpu.sync_copy(p0_hbm.at[pl.ds(off, CCH)], i0_v)
        pltpu.sync_copy(p1_hbm.at[pl.ds(off, CCH)], i1_v)
        cp0 = pltpu.async_copy(ys_hbm.at[i0_v], r0_v, s0)
        cp1 = pltpu.async_copy(ys_hbm.at[i1_v], r1_v, s1)
        cp0.wait()
        cp1.wait()

        def row(i, rc):
            for j in range(D_MODEL // 16):
                sl = pl.ds(j * 16, 16)
                r0_v[i, sl] = r0_v[i, sl] + r1_v[i, sl]
            return rc

        lax.fori_loop(0, CCH, row, 0)
        pltpu.sync_copy(r0_v, out_hbm.at[pl.ds(off, CCH)])
        return carry

    lax.fori_loop(0, TPW // CCH, chunk, 0)


@functools.cache
def _combine_rows():
    return functools.partial(
        pl.kernel, mesh=_sc_mesh(),
        out_type=jax.ShapeDtypeStruct((T, D_MODEL), jnp.float32),
        scratch_types=[
            pltpu.VMEM((CCH,), jnp.int32),
            pltpu.VMEM((CCH,), jnp.int32),
            pltpu.VMEM((CCH, D_MODEL), jnp.float32),
            pltpu.VMEM((CCH, D_MODEL), jnp.float32),
            pltpu.SemaphoreType.DMA,
            pltpu.SemaphoreType.DMA,
        ],
    )(_combine_rows_body)


def kernel(x, Wg, W1, W3, W2):
    b, s, d = x.shape
    xf = x.reshape(-1, d)

    logits = xf @ Wg
    gates = jax.nn.softmax(logits, axis=-1)
    tg, ti = lax.top_k(gates, TOPK)
    tg = tg / jnp.sum(tg, axis=-1, keepdims=True)

    eid = ti.reshape(-1).astype(jnp.int32)
    oh = (eid[:, None] == jnp.arange(NE, dtype=jnp.int32)[None, :]).astype(jnp.int32)
    within = jnp.cumsum(oh, axis=0) - oh
    rank = jnp.take_along_axis(within, eid[:, None], axis=1)[:, 0]
    counts = jnp.sum(oh, axis=0)
    padded = ((counts + BT - 1) // BT) * BT
    cumpad = jnp.cumsum(padded)
    offs = cumpad - padded
    dst = offs[eid] + rank
    total = cumpad[-1]
    n_live = total // BT

    tile_starts = jnp.arange(N_TILES, dtype=jnp.int32) * BT
    te = jnp.searchsorted(cumpad, tile_starts, side="right").astype(jnp.int32)
    te_lastlive = jnp.take(te, jnp.maximum(n_live - 1, 0))
    te = jnp.where(tile_starts < total, te, te_lastlive)

    row_token = (jnp.arange(N_MAX, dtype=jnp.int32) % T).at[dst].set(
        jnp.arange(NP, dtype=jnp.int32) // TOPK)
    row_gate = jnp.zeros((N_MAX,), jnp.float32).at[dst].set(tg.reshape(-1))
    gate_tiles = row_gate.reshape(N_TILES, 1, BT)
    pos = dst.reshape(T, TOPK)
    p0 = pos[:, 0] + 0
    p1 = pos[:, 1] + 0

    nl = jnp.asarray(n_live, jnp.int32).reshape(1)

    xs = _gather_rows()(xf, row_token)
    ys = _ffn(te, nl, xs, gate_tiles, W1, W3, W2)
    outf = _combine_rows()(ys, p0, p1)

    return outf.reshape(b, s, d), jnp.asarray(0.0, x.dtype)


# speedup vs baseline: 1.1649x; 1.1649x over previous
import functools
import jax
import jax.numpy as jnp
from jax import lax
from jax.experimental import pallas as pl
from jax.experimental.pallas import tpu as pltpu
from jax.experimental.pallas import tpu_sc as plsc

D_MODEL = 768
D_FF = 2048
NE = 8
TOPK = 2
T = 2048
NP = T * TOPK
BT = 256
N_TILES = NP // BT + NE
N_MAX = N_TILES * BT
NW = 32
GPW = N_MAX // NW
GCH = 64
TPW = T // NW
CCH = 32


def _ffn_tile(te_ref, nl_ref, xs_ref, w1_ref, w3_ref, w2_ref, out_ref):
    i = pl.program_id(0)

    @pl.when(i < nl_ref[0])
    def _():
        xv = xs_ref[...]
        h = jnp.dot(xv, w1_ref[0], preferred_element_type=jnp.float32)
        g = jnp.dot(xv, w3_ref[0], preferred_element_type=jnp.float32)
        a = (h * jax.nn.sigmoid(h)) * g
        out_ref[...] = jnp.dot(a, w2_ref[0], preferred_element_type=jnp.float32)


def _ffn(te, nl, xs, W1, W3, W2):
    grid_spec = pltpu.PrefetchScalarGridSpec(
        num_scalar_prefetch=2,
        grid=(N_TILES,),
        in_specs=[
            pl.BlockSpec((BT, D_MODEL), lambda i, te, nl: (i, 0)),
            pl.BlockSpec((1, D_MODEL, D_FF), lambda i, te, nl: (te[i], 0, 0)),
            pl.BlockSpec((1, D_MODEL, D_FF), lambda i, te, nl: (te[i], 0, 0)),
            pl.BlockSpec((1, D_FF, D_MODEL), lambda i, te, nl: (te[i], 0, 0)),
        ],
        out_specs=pl.BlockSpec((BT, D_MODEL), lambda i, te, nl: (i, 0)),
    )
    return pl.pallas_call(
        _ffn_tile,
        grid_spec=grid_spec,
        out_shape=jax.ShapeDtypeStruct((N_MAX, D_MODEL), jnp.float32),
        compiler_params=pltpu.CompilerParams(
            dimension_semantics=("arbitrary",)),
    )(te, nl, xs, W1, W3, W2)


@functools.cache
def _sc_mesh():
    return plsc.VectorSubcoreMesh(
        core_axis_name="c", subcore_axis_name="s", num_cores=2)


def _scatter_rows_body(x_hbm, p0_hbm, p1_hbm, out_hbm, i0_v, i1_v, rows_v, s0, s1, sr):
    wid = lax.axis_index("s") * 2 + lax.axis_index("c")
    base = wid * TPW
    pltpu.sync_copy(p0_hbm.at[pl.ds(base, TPW)], i0_v)
    pltpu.sync_copy(p1_hbm.at[pl.ds(base, TPW)], i1_v)
    pltpu.async_copy(x_hbm.at[pl.ds(base, TPW)], rows_v, sr).wait()
    c0 = pltpu.async_copy(rows_v, out_hbm.at[i0_v], s0)
    c1 = pltpu.async_copy(rows_v, out_hbm.at[i1_v], s1)
    c0.wait()
    c1.wait()


@functools.cache
def _scatter_rows():
    return functools.partial(
        pl.kernel, mesh=_sc_mesh(),
        out_type=jax.ShapeDtypeStruct((N_MAX, D_MODEL), jnp.float32),
        scratch_types=[
            pltpu.VMEM((TPW,), jnp.int32),
            pltpu.VMEM((TPW,), jnp.int32),
            pltpu.VMEM((TPW, D_MODEL), jnp.float32),
            pltpu.SemaphoreType.DMA,
            pltpu.SemaphoreType.DMA,
            pltpu.SemaphoreType.DMA,
        ],
    )(_scatter_rows_body)


def _combine_rows_body(ys_hbm, p0_hbm, p1_hbm, g0_hbm, g1_hbm, out_hbm,
                       i0_v, i1_v, g0_v, g1_v, r0_v, r1_v, s0, s1):
    wid = lax.axis_index("s") * 2 + lax.axis_index("c")
    base = wid * TPW

    def chunk(c, carry):
        off = base + c * CCH
        pltpu.sync_copy(p0_hbm.at[pl.ds(off, CCH)], i0_v)
        pltpu.sync_copy(p1_hbm.at[pl.ds(off, CCH)], i1_v)
        cp0 = pltpu.async_copy(ys_hbm.at[i0_v], r0_v, s0)
        cp1 = pltpu.async_copy(ys_hbm.at[i1_v], r1_v, s1)
        pltpu.sync_copy(g0_hbm.at[pl.ds(off, CCH)], g0_v)
        pltpu.sync_copy(g1_hbm.at[pl.ds(off, CCH)], g1_v)
        cp0.wait()
        cp1.wait()

        def row(i, rc):
            gv0 = g0_v[i, :]
            gv1 = g1_v[i, :]
            for j in range(D_MODEL // 16):
                sl = pl.ds(j * 16, 16)
                r0_v[i, sl] = gv0 * r0_v[i, sl] + gv1 * r1_v[i, sl]
            return rc

        lax.fori_loop(0, CCH, row, 0)
        pltpu.sync_copy(r0_v, out_hbm.at[pl.ds(off, CCH)])
        return carry

    lax.fori_loop(0, TPW // CCH, chunk, 0)


@functools.cache
def _combine_rows():
    return functools.partial(
        pl.kernel, mesh=_sc_mesh(),
        out_type=jax.ShapeDtypeStruct((T, D_MODEL), jnp.float32),
        scratch_types=[
            pltpu.VMEM((CCH,), jnp.int32),
            pltpu.VMEM((CCH,), jnp.int32),
            pltpu.VMEM((CCH, 16), jnp.float32),
            pltpu.VMEM((CCH, 16), jnp.float32),
            pltpu.VMEM((CCH, D_MODEL), jnp.float32),
            pltpu.VMEM((CCH, D_MODEL), jnp.float32),
            pltpu.SemaphoreType.DMA,
            pltpu.SemaphoreType.DMA,
        ],
    )(_combine_rows_body)


def kernel(x, Wg, W1, W3, W2):
    b, s, d = x.shape
    xf = x.reshape(-1, d)

    logits = xf @ Wg
    gates = jax.nn.softmax(logits, axis=-1)
    tg, ti = lax.top_k(gates, TOPK)
    tg = tg / jnp.sum(tg, axis=-1, keepdims=True)

    eid = ti.reshape(-1).astype(jnp.int32)
    oh = (eid[:, None] == jnp.arange(NE, dtype=jnp.int32)[None, :]).astype(jnp.int32)
    within = jnp.cumsum(oh, axis=0) - oh
    rank = jnp.take_along_axis(within, eid[:, None], axis=1)[:, 0]
    counts = jnp.sum(oh, axis=0)
    padded = ((counts + BT - 1) // BT) * BT
    cumpad = jnp.cumsum(padded)
    offs = cumpad - padded
    dst = offs[eid] + rank
    total = cumpad[-1]
    n_live = total // BT

    tile_starts = jnp.arange(N_TILES, dtype=jnp.int32) * BT
    te = jnp.searchsorted(cumpad, tile_starts, side="right").astype(jnp.int32)
    te_lastlive = jnp.take(te, jnp.maximum(n_live - 1, 0))
    te = jnp.where(tile_starts < total, te, te_lastlive)

    pos = dst.reshape(T, TOPK)
    p0 = pos[:, 0] + 0
    p1 = pos[:, 1] + 0
    g0b = jnp.broadcast_to(tg[:, 0:1], (T, 16)) + 0.0
    g1b = jnp.broadcast_to(tg[:, 1:2], (T, 16)) + 0.0

    nl = jnp.asarray(n_live, jnp.int32).reshape(1)

    xs = _scatter_rows()(xf, p0, p1)
    ys = _ffn(te, nl, xs, W1, W3, W2)
    outf = _combine_rows()(ys, p0, p1, g0b, g1b)

    return outf.reshape(b, s, d), jnp.asarray(0.0, x.dtype)


# speedup vs baseline: 1.1656x; 1.0006x over previous
import functools
import jax
import jax.numpy as jnp
from jax import lax
from jax.experimental import pallas as pl
from jax.experimental.pallas import tpu as pltpu
from jax.experimental.pallas import tpu_sc as plsc

D_MODEL = 768
D_FF = 2048
NE = 8
TOPK = 2
T = 2048
NP = T * TOPK
BT = 256
N_TILES = NP // BT + NE
N_MAX = N_TILES * BT
NW = 32
GPW = N_MAX // NW
GCH = 64
TPW = T // NW
CCH = 32


def _ffn_tile(te_ref, nl_ref, xs_ref, w1_ref, w3_ref, w2_ref, out_ref):
    i = pl.program_id(0)

    @pl.when(i < nl_ref[0])
    def _():
        xv = xs_ref[...]
        h = jnp.dot(xv, w1_ref[0], preferred_element_type=jnp.float32)
        g = jnp.dot(xv, w3_ref[0], preferred_element_type=jnp.float32)
        a = (h * jax.nn.sigmoid(h)) * g
        out_ref[...] = jnp.dot(a, w2_ref[0], preferred_element_type=jnp.float32)


def _ffn(te, nl, xs, W1, W3, W2):
    grid_spec = pltpu.PrefetchScalarGridSpec(
        num_scalar_prefetch=2,
        grid=(N_TILES,),
        in_specs=[
            pl.BlockSpec((BT, D_MODEL), lambda i, te, nl: (i, 0)),
            pl.BlockSpec((1, D_MODEL, D_FF), lambda i, te, nl: (te[i], 0, 0)),
            pl.BlockSpec((1, D_MODEL, D_FF), lambda i, te, nl: (te[i], 0, 0)),
            pl.BlockSpec((1, D_FF, D_MODEL), lambda i, te, nl: (te[i], 0, 0)),
        ],
        out_specs=pl.BlockSpec((BT, D_MODEL), lambda i, te, nl: (i, 0)),
    )
    return pl.pallas_call(
        _ffn_tile,
        grid_spec=grid_spec,
        out_shape=jax.ShapeDtypeStruct((N_MAX, D_MODEL), jnp.float32),
        compiler_params=pltpu.CompilerParams(
            dimension_semantics=("arbitrary",)),
    )(te, nl, xs, W1, W3, W2)


@functools.cache
def _sc_mesh():
    return plsc.VectorSubcoreMesh(
        core_axis_name="c", subcore_axis_name="s", num_cores=2)


def _scatter_rows_body(x_hbm, p0_hbm, p1_hbm, out_hbm, i0_v, i1_v, rows_v, s0, s1, sr):
    wid = lax.axis_index("s") * 2 + lax.axis_index("c")
    base = wid * TPW
    pltpu.sync_copy(p0_hbm.at[pl.ds(base, TPW)], i0_v)
    pltpu.sync_copy(p1_hbm.at[pl.ds(base, TPW)], i1_v)
    pltpu.async_copy(x_hbm.at[pl.ds(base, TPW)], rows_v, sr).wait()
    c0 = pltpu.async_copy(rows_v, out_hbm.at[i0_v], s0)
    c1 = pltpu.async_copy(rows_v, out_hbm.at[i1_v], s1)
    c0.wait()
    c1.wait()


@functools.cache
def _scatter_rows():
    return functools.partial(
        pl.kernel, mesh=_sc_mesh(),
        out_type=jax.ShapeDtypeStruct((N_MAX, D_MODEL), jnp.float32),
        scratch_types=[
            pltpu.VMEM((TPW,), jnp.int32),
            pltpu.VMEM((TPW,), jnp.int32),
            pltpu.VMEM((TPW, D_MODEL), jnp.float32),
            pltpu.SemaphoreType.DMA,
            pltpu.SemaphoreType.DMA,
            pltpu.SemaphoreType.DMA,
        ],
    )(_scatter_rows_body)


def _combine_rows_body(ys_hbm, p0_hbm, p1_hbm, g0_hbm, g1_hbm, out_hbm,
                       i0_v, i1_v, g0_v, g1_v, r0_v, r1_v, s0, s1):
    wid = lax.axis_index("s") * 2 + lax.axis_index("c")
    base = wid * TPW

    def chunk(c, carry):
        off = base + c * CCH
        pltpu.sync_copy(p0_hbm.at[pl.ds(off, CCH)], i0_v)
        pltpu.sync_copy(p1_hbm.at[pl.ds(off, CCH)], i1_v)
        cp0 = pltpu.async_copy(ys_hbm.at[i0_v], r0_v, s0)
        cp1 = pltpu.async_copy(ys_hbm.at[i1_v], r1_v, s1)
        pltpu.sync_copy(g0_hbm.at[pl.ds(off, CCH)], g0_v)
        pltpu.sync_copy(g1_hbm.at[pl.ds(off, CCH)], g1_v)
        cp0.wait()
        cp1.wait()

        def row(i, rc):
            gv0 = g0_v[i, :]
            gv1 = g1_v[i, :]
            for j in range(D_MODEL // 16):
                sl = pl.ds(j * 16, 16)
                r0_v[i, sl] = gv0 * r0_v[i, sl] + gv1 * r1_v[i, sl]
            return rc

        lax.fori_loop(0, CCH, row, 0)
        pltpu.sync_copy(r0_v, out_hbm.at[pl.ds(off, CCH)])
        return carry

    lax.fori_loop(0, TPW // CCH, chunk, 0)


@functools.cache
def _combine_rows():
    return functools.partial(
        pl.kernel, mesh=_sc_mesh(),
        out_type=jax.ShapeDtypeStruct((T, D_MODEL), jnp.float32),
        scratch_types=[
            pltpu.VMEM((CCH,), jnp.int32),
            pltpu.VMEM((CCH,), jnp.int32),
            pltpu.VMEM((CCH, 16), jnp.float32),
            pltpu.VMEM((CCH, 16), jnp.float32),
            pltpu.VMEM((CCH, D_MODEL), jnp.float32),
            pltpu.VMEM((CCH, D_MODEL), jnp.float32),
            pltpu.SemaphoreType.DMA,
            pltpu.SemaphoreType.DMA,
        ],
    )(_combine_rows_body)


def kernel(x, Wg, W1, W3, W2):
    b, s, d = x.shape
    xf = x.reshape(-1, d)

    logits = xf @ Wg
    lane8 = jnp.arange(NE, dtype=jnp.int32)[None, :]
    i0 = jnp.argmax(logits, axis=1).astype(jnp.int32)
    l0 = jnp.max(logits, axis=1)
    masked = jnp.where(lane8 == i0[:, None], -jnp.inf, logits)
    i1 = jnp.argmax(masked, axis=1).astype(jnp.int32)
    l1 = jnp.max(masked, axis=1)
    g0 = 1.0 / (1.0 + jnp.exp(l1 - l0))
    tg = jnp.stack([g0, 1.0 - g0], axis=1)
    ti = jnp.stack([i0, i1], axis=1)

    eid = ti.reshape(-1).astype(jnp.int32)
    oh = (eid[:, None] == jnp.arange(NE, dtype=jnp.int32)[None, :]).astype(jnp.int32)
    within = jnp.cumsum(oh, axis=0) - oh
    rank = jnp.take_along_axis(within, eid[:, None], axis=1)[:, 0]
    counts = jnp.sum(oh, axis=0)
    padded = ((counts + BT - 1) // BT) * BT
    cumpad = jnp.cumsum(padded)
    offs = cumpad - padded
    dst = offs[eid] + rank
    total = cumpad[-1]
    n_live = total // BT

    tile_starts = jnp.arange(N_TILES, dtype=jnp.int32) * BT
    te = jnp.searchsorted(cumpad, tile_starts, side="right").astype(jnp.int32)
    te_lastlive = jnp.take(te, jnp.maximum(n_live - 1, 0))
    te = jnp.where(tile_starts < total, te, te_lastlive)

    pos = dst.reshape(T, TOPK)
    p0 = pos[:, 0] + 0
    p1 = pos[:, 1] + 0
    g0b = jnp.broadcast_to(tg[:, 0:1], (T, 16)) + 0.0
    g1b = jnp.broadcast_to(tg[:, 1:2], (T, 16)) + 0.0

    nl = jnp.asarray(n_live, jnp.int32).reshape(1)

    xs = _scatter_rows()(xf, p0, p1)
    ys = _ffn(te, nl, xs, W1, W3, W2)
    outf = _combine_rows()(ys, p0, p1, g0b, g1b)

    return outf.reshape(b, s, d), jnp.asarray(0.0, x.dtype)


# speedup vs baseline: 1.2604x; 1.0814x over previous
import functools
import jax
import jax.numpy as jnp
from jax import lax
from jax.experimental import pallas as pl
from jax.experimental.pallas import tpu as pltpu
from jax.experimental.pallas import tpu_sc as plsc

D_MODEL = 768
D_FF = 2048
NE = 8
TOPK = 2
T = 2048
NP = T * TOPK
BT = 256
N_TILES = NP // BT + NE
N_MAX = N_TILES * BT
NW = 32
GPW = N_MAX // NW
GCH = 64
TPW = T // NW
CCH = 32


RCH = 256
N_TE_PAD = 128


def _router_body(x_ref, wg_ref, p0_ref, p1_ref, g0_ref, g1_ref, te_ref):
    xv = x_ref[...]
    wg = wg_ref[...]
    logits = jnp.dot(xv, wg, preferred_element_type=jnp.float32)
    lane = lax.broadcasted_iota(jnp.int32, (T, NE), 1)
    l0 = jnp.max(logits, axis=1, keepdims=True)
    i0 = jnp.min(jnp.where(logits == l0, lane, NE), axis=1, keepdims=True)
    masked = jnp.where(lane == i0, jnp.float32(-1e30), logits)
    l1 = jnp.max(masked, axis=1, keepdims=True)
    i1 = jnp.min(jnp.where(masked == l1, lane, NE), axis=1, keepdims=True)
    g0 = 1.0 / (1.0 + jnp.exp(l1 - l0))

    r = lax.broadcasted_iota(jnp.int32, (RCH, RCH), 0)
    c = lax.broadcasted_iota(jnp.int32, (RCH, RCH), 1)
    tril = (r >= c).astype(jnp.float32)
    lane_c = lax.broadcasted_iota(jnp.int32, (RCH, NE), 1)

    run = jnp.zeros((1, NE), jnp.float32)
    cnts, incls, runs = [], [], []
    for ci in range(T // RCH):
        s = ci * RCH
        i0c = lax.slice(i0, (s, 0), (s + RCH, 1))
        i1c = lax.slice(i1, (s, 0), (s + RCH, 1))
        cnt = ((lane_c == i0c).astype(jnp.float32)
               + (lane_c == i1c).astype(jnp.float32))
        inc = jnp.dot(tril, cnt, preferred_element_type=jnp.float32)
        cnts.append(cnt)
        incls.append(inc)
        runs.append(run)
        run = run + lax.slice(inc, (RCH - 1, 0), (RCH, NE))

    counts_i = run.astype(jnp.int32)
    padded = (((counts_i + (BT - 1)) // BT) * BT).astype(jnp.float32)
    r8 = lax.broadcasted_iota(jnp.int32, (NE, NE), 0)
    c8 = lax.broadcasted_iota(jnp.int32, (NE, NE), 1)
    upper8 = (r8 <= c8).astype(jnp.float32)
    cumpad = jnp.dot(padded, upper8, preferred_element_type=jnp.float32)
    offs = cumpad - padded

    for ci in range(T // RCH):
        s = ci * RCH
        pref = incls[ci] - cnts[ci] + runs[ci] + offs
        i0c = lax.slice(i0, (s, 0), (s + RCH, 1))
        i1c = lax.slice(i1, (s, 0), (s + RCH, 1))
        p0c = jnp.sum(jnp.where(lane_c == i0c, pref, 0.0), axis=1)
        p1c = jnp.sum(jnp.where(lane_c == i1c, pref, 0.0), axis=1)
        p0_ref[ci, :] = p0c.astype(jnp.int32)
        p1_ref[ci, :] = p1c.astype(jnp.int32)
        g0c = lax.slice(g0, (s, 0), (s + RCH, 1))
        g0_ref[pl.ds(s, RCH), :] = jnp.broadcast_to(g0c, (RCH, 16))
        g1_ref[pl.ds(s, RCH), :] = jnp.broadcast_to(1.0 - g0c, (RCH, 16))

    starts = (lax.broadcasted_iota(jnp.int32, (1, N_TE_PAD), 1)
              * BT).astype(jnp.float32)
    te = jnp.zeros((1, N_TE_PAD), jnp.float32)
    for e in range(NE):
        ce = lax.slice(cumpad, (0, e), (1, e + 1))
        te = te + (starts >= jnp.broadcast_to(ce, (1, N_TE_PAD))).astype(jnp.float32)
    te = jnp.minimum(te, jnp.float32(NE - 1))
    total = lax.slice(cumpad, (0, NE - 1), (1, NE))
    nl = jnp.broadcast_to(total * (1.0 / BT), (1, N_TE_PAD))
    ilane = lax.broadcasted_iota(jnp.int32, (1, N_TE_PAD), 1)
    te_ref[...] = jnp.where(ilane == 64, nl, te).astype(jnp.int32)


def _router(xf, Wg, interpret=False):
    return pl.pallas_call(
        _router_body,
        out_shape=(
            jax.ShapeDtypeStruct((T // RCH, RCH), jnp.int32),
            jax.ShapeDtypeStruct((T // RCH, RCH), jnp.int32),
            jax.ShapeDtypeStruct((T, 16), jnp.float32),
            jax.ShapeDtypeStruct((T, 16), jnp.float32),
            jax.ShapeDtypeStruct((1, N_TE_PAD), jnp.int32),
        ),
        interpret=interpret,
    )(xf, Wg)


def _ffn_tile(te_ref, nl_ref, xs_ref, w1_ref, w3_ref, w2_ref, out_ref):
    i = pl.program_id(0)

    @pl.when(i < nl_ref[0])
    def _():
        xv = xs_ref[...]
        h = jnp.dot(xv, w1_ref[0], preferred_element_type=jnp.float32)
        g = jnp.dot(xv, w3_ref[0], preferred_element_type=jnp.float32)
        a = (h * jax.nn.sigmoid(h)) * g
        out_ref[...] = jnp.dot(a, w2_ref[0], preferred_element_type=jnp.float32)


def _ffn(te, nl, xs, W1, W3, W2):
    grid_spec = pltpu.PrefetchScalarGridSpec(
        num_scalar_prefetch=2,
        grid=(N_TILES,),
        in_specs=[
            pl.BlockSpec((BT, D_MODEL), lambda i, te, nl: (i, 0)),
            pl.BlockSpec((1, D_MODEL, D_FF), lambda i, te, nl: (te[i], 0, 0)),
            pl.BlockSpec((1, D_MODEL, D_FF), lambda i, te, nl: (te[i], 0, 0)),
            pl.BlockSpec((1, D_FF, D_MODEL), lambda i, te, nl: (te[i], 0, 0)),
        ],
        out_specs=pl.BlockSpec((BT, D_MODEL), lambda i, te, nl: (i, 0)),
    )
    return pl.pallas_call(
        _ffn_tile,
        grid_spec=grid_spec,
        out_shape=jax.ShapeDtypeStruct((N_MAX, D_MODEL), jnp.float32),
        compiler_params=pltpu.CompilerParams(
            dimension_semantics=("arbitrary",)),
    )(te, nl, xs, W1, W3, W2)


@functools.cache
def _sc_mesh():
    return plsc.VectorSubcoreMesh(
        core_axis_name="c", subcore_axis_name="s", num_cores=2)


def _scatter_rows_body(x_hbm, p0_hbm, p1_hbm, out_hbm, i0_v, i1_v, rows_v, s0, s1, sr):
    wid = lax.axis_index("s") * 2 + lax.axis_index("c")
    base = wid * TPW
    pltpu.sync_copy(p0_hbm.at[pl.ds(base, TPW)], i0_v)
    pltpu.sync_copy(p1_hbm.at[pl.ds(base, TPW)], i1_v)
    pltpu.async_copy(x_hbm.at[pl.ds(base, TPW)], rows_v, sr).wait()
    c0 = pltpu.async_copy(rows_v, out_hbm.at[i0_v], s0)
    c1 = pltpu.async_copy(rows_v, out_hbm.at[i1_v], s1)
    c0.wait()
    c1.wait()


@functools.cache
def _scatter_rows():
    return functools.partial(
        pl.kernel, mesh=_sc_mesh(),
        out_type=jax.ShapeDtypeStruct((N_MAX, D_MODEL), jnp.float32),
        scratch_types=[
            pltpu.VMEM((TPW,), jnp.int32),
            pltpu.VMEM((TPW,), jnp.int32),
            pltpu.VMEM((TPW, D_MODEL), jnp.float32),
            pltpu.SemaphoreType.DMA,
            pltpu.SemaphoreType.DMA,
            pltpu.SemaphoreType.DMA,
        ],
    )(_scatter_rows_body)


def _combine_rows_body(ys_hbm, p0_hbm, p1_hbm, g0_hbm, g1_hbm, out_hbm,
                       i0_v, i1_v, g0_v, g1_v, r0_v, r1_v, s0, s1):
    wid = lax.axis_index("s") * 2 + lax.axis_index("c")
    base = wid * TPW

    def chunk(c, carry):
        off = base + c * CCH
        pltpu.sync_copy(p0_hbm.at[pl.ds(off, CCH)], i0_v)
        pltpu.sync_copy(p1_hbm.at[pl.ds(off, CCH)], i1_v)
        cp0 = pltpu.async_copy(ys_hbm.at[i0_v], r0_v, s0)
        cp1 = pltpu.async_copy(ys_hbm.at[i1_v], r1_v, s1)
        pltpu.sync_copy(g0_hbm.at[pl.ds(off, CCH)], g0_v)
        pltpu.sync_copy(g1_hbm.at[pl.ds(off, CCH)], g1_v)
        cp0.wait()
        cp1.wait()

        def row(i, rc):
            gv0 = g0_v[i, :]
            gv1 = g1_v[i, :]
            for j in range(D_MODEL // 16):
                sl = pl.ds(j * 16, 16)
                r0_v[i, sl] = gv0 * r0_v[i, sl] + gv1 * r1_v[i, sl]
            return rc

        lax.fori_loop(0, CCH, row, 0)
        pltpu.sync_copy(r0_v, out_hbm.at[pl.ds(off, CCH)])
        return carry

    lax.fori_loop(0, TPW // CCH, chunk, 0)


@functools.cache
def _combine_rows():
    return functools.partial(
        pl.kernel, mesh=_sc_mesh(),
        out_type=jax.ShapeDtypeStruct((T, D_MODEL), jnp.float32),
        scratch_types=[
            pltpu.VMEM((CCH,), jnp.int32),
            pltpu.VMEM((CCH,), jnp.int32),
            pltpu.VMEM((CCH, 16), jnp.float32),
            pltpu.VMEM((CCH, 16), jnp.float32),
            pltpu.VMEM((CCH, D_MODEL), jnp.float32),
            pltpu.VMEM((CCH, D_MODEL), jnp.float32),
            pltpu.SemaphoreType.DMA,
            pltpu.SemaphoreType.DMA,
        ],
    )(_combine_rows_body)


def kernel(x, Wg, W1, W3, W2):
    b, s, d = x.shape
    xf = x.reshape(-1, d)

    p0m, p1m, g0b, g1b, tem = _router(xf, Wg)
    p0 = p0m.reshape(T)
    p1 = p1m.reshape(T)
    te = tem[0, :N_TILES]
    nl = tem[0, 64:65]

    xs = _scatter_rows()(xf, p0, p1)
    ys = _ffn(te, nl, xs, W1, W3, W2)
    outf = _combine_rows()(ys, p0, p1, g0b, g1b)

    return outf.reshape(b, s, d), jnp.asarray(0.0, x.dtype)


# speedup vs baseline: 1.2642x; 1.0030x over previous
import functools
import jax
import jax.numpy as jnp
from jax import lax
from jax.experimental import pallas as pl
from jax.experimental.pallas import tpu as pltpu
from jax.experimental.pallas import tpu_sc as plsc

D_MODEL = 768
D_FF = 2048
NE = 8
TOPK = 2
T = 2048
NP = T * TOPK
BT = 256
N_TILES = NP // BT + NE
N_MAX = N_TILES * BT
NW = 32
GPW = N_MAX // NW
GCH = 64
TPW = T // NW
CCH = 32


RCH = 256
N_TE_PAD = 128


def _router_body(x_ref, wg_ref, p0_ref, p1_ref, g0_ref, g1_ref, te_ref):
    xv = x_ref[...]
    wg = wg_ref[...]
    logits = jnp.dot(xv, wg, preferred_element_type=jnp.float32)
    lane = lax.broadcasted_iota(jnp.int32, (T, NE), 1)
    l0 = jnp.max(logits, axis=1, keepdims=True)
    i0 = jnp.min(jnp.where(logits == l0, lane, NE), axis=1, keepdims=True)
    masked = jnp.where(lane == i0, jnp.float32(-1e30), logits)
    l1 = jnp.max(masked, axis=1, keepdims=True)
    i1 = jnp.min(jnp.where(masked == l1, lane, NE), axis=1, keepdims=True)
    g0 = 1.0 / (1.0 + jnp.exp(l1 - l0))

    r = lax.broadcasted_iota(jnp.int32, (RCH, RCH), 0)
    c = lax.broadcasted_iota(jnp.int32, (RCH, RCH), 1)
    tril = (r >= c).astype(jnp.float32)
    lane_c = lax.broadcasted_iota(jnp.int32, (RCH, NE), 1)

    run = jnp.zeros((1, NE), jnp.float32)
    cnts, incls, runs = [], [], []
    for ci in range(T // RCH):
        s = ci * RCH
        i0c = lax.slice(i0, (s, 0), (s + RCH, 1))
        i1c = lax.slice(i1, (s, 0), (s + RCH, 1))
        cnt = ((lane_c == i0c).astype(jnp.float32)
               + (lane_c == i1c).astype(jnp.float32))
        inc = jnp.dot(tril, cnt, preferred_element_type=jnp.float32)
        cnts.append(cnt)
        incls.append(inc)
        runs.append(run)
        run = run + lax.slice(inc, (RCH - 1, 0), (RCH, NE))

    counts_i = run.astype(jnp.int32)
    padded = (((counts_i + (BT - 1)) // BT) * BT).astype(jnp.float32)
    r8 = lax.broadcasted_iota(jnp.int32, (NE, NE), 0)
    c8 = lax.broadcasted_iota(jnp.int32, (NE, NE), 1)
    upper8 = (r8 <= c8).astype(jnp.float32)
    cumpad = jnp.dot(padded, upper8, preferred_element_type=jnp.float32)
    offs = cumpad - padded

    for ci in range(T // RCH):
        s = ci * RCH
        pref = incls[ci] - cnts[ci] + runs[ci] + offs
        i0c = lax.slice(i0, (s, 0), (s + RCH, 1))
        i1c = lax.slice(i1, (s, 0), (s + RCH, 1))
        p0c = jnp.sum(jnp.where(lane_c == i0c, pref, 0.0), axis=1)
        p1c = jnp.sum(jnp.where(lane_c == i1c, pref, 0.0), axis=1)
        p0_ref[ci, :] = p0c.astype(jnp.int32)
        p1_ref[ci, :] = p1c.astype(jnp.int32)
        g0c = lax.slice(g0, (s, 0), (s + RCH, 1))
        g0_ref[pl.ds(s, RCH), :] = jnp.broadcast_to(g0c, (RCH, 16))
        g1_ref[pl.ds(s, RCH), :] = jnp.broadcast_to(1.0 - g0c, (RCH, 16))

    starts = (lax.broadcasted_iota(jnp.int32, (1, N_TE_PAD), 1)
              * BT).astype(jnp.float32)
    te = jnp.zeros((1, N_TE_PAD), jnp.float32)
    for e in range(NE):
        ce = lax.slice(cumpad, (0, e), (1, e + 1))
        te = te + (starts >= jnp.broadcast_to(ce, (1, N_TE_PAD))).astype(jnp.float32)
    te = jnp.minimum(te, jnp.float32(NE - 1))
    total = lax.slice(cumpad, (0, NE - 1), (1, NE))
    nl = jnp.broadcast_to(total * (1.0 / BT), (1, N_TE_PAD))
    ilane = lax.broadcasted_iota(jnp.int32, (1, N_TE_PAD), 1)
    te_ref[...] = jnp.where(ilane == 64, nl, te).astype(jnp.int32)


def _router(xf, Wg):
    return pl.pallas_call(
        _router_body,
        out_shape=(
            jax.ShapeDtypeStruct((T // RCH, RCH), jnp.int32),
            jax.ShapeDtypeStruct((T // RCH, RCH), jnp.int32),
            jax.ShapeDtypeStruct((T, 16), jnp.float32),
            jax.ShapeDtypeStruct((T, 16), jnp.float32),
            jax.ShapeDtypeStruct((1, N_TE_PAD), jnp.int32),
        ),
    )(xf, Wg)


def _ffn_tile(te_ref, nl_ref, xs_ref, w1_ref, w3_ref, w2_ref, out_ref):
    i = pl.program_id(0)

    @pl.when(i < nl_ref[0])
    def _():
        xv = xs_ref[...]
        h = jnp.dot(xv, w1_ref[0], preferred_element_type=jnp.float32)
        g = jnp.dot(xv, w3_ref[0], preferred_element_type=jnp.float32)
        a = (h * jax.nn.sigmoid(h)) * g
        out_ref[...] = jnp.dot(a, w2_ref[0], preferred_element_type=jnp.float32)


def _ffn(te, nl, xs, W1, W3, W2):
    grid_spec = pltpu.PrefetchScalarGridSpec(
        num_scalar_prefetch=2,
        grid=(N_TILES,),
        in_specs=[
            pl.BlockSpec((BT, D_MODEL), lambda i, te, nl: (i, 0)),
            pl.BlockSpec((1, D_MODEL, D_FF), lambda i, te, nl: (te[i], 0, 0)),
            pl.BlockSpec((1, D_MODEL, D_FF), lambda i, te, nl: (te[i], 0, 0)),
            pl.BlockSpec((1, D_FF, D_MODEL), lambda i, te, nl: (te[i], 0, 0)),
        ],
        out_specs=pl.BlockSpec((BT, D_MODEL), lambda i, te, nl: (i, 0)),
    )
    return pl.pallas_call(
        _ffn_tile,
        grid_spec=grid_spec,
        out_shape=jax.ShapeDtypeStruct((N_MAX, D_MODEL), jnp.float32),
        compiler_params=pltpu.CompilerParams(
            dimension_semantics=("arbitrary",)),
    )(te, nl, xs, W1, W3, W2)


@functools.cache
def _sc_mesh():
    return plsc.VectorSubcoreMesh(
        core_axis_name="c", subcore_axis_name="s", num_cores=2)


def _scatter_rows_body(x_hbm, p0_hbm, p1_hbm, out_hbm, i0_v, i1_v, rows_v, s0, s1, sr):
    wid = lax.axis_index("s") * 2 + lax.axis_index("c")
    base = wid * TPW
    pltpu.sync_copy(p0_hbm.at[pl.ds(base, TPW)], i0_v)
    pltpu.sync_copy(p1_hbm.at[pl.ds(base, TPW)], i1_v)
    pltpu.async_copy(x_hbm.at[pl.ds(base, TPW)], rows_v, sr).wait()
    c0 = pltpu.async_copy(rows_v, out_hbm.at[i0_v], s0)
    c1 = pltpu.async_copy(rows_v, out_hbm.at[i1_v], s1)
    c0.wait()
    c1.wait()


@functools.cache
def _scatter_rows():
    return functools.partial(
        pl.kernel, mesh=_sc_mesh(),
        out_type=jax.ShapeDtypeStruct((N_MAX, D_MODEL), jnp.float32),
        scratch_types=[
            pltpu.VMEM((TPW,), jnp.int32),
            pltpu.VMEM((TPW,), jnp.int32),
            pltpu.VMEM((TPW, D_MODEL), jnp.float32),
            pltpu.SemaphoreType.DMA,
            pltpu.SemaphoreType.DMA,
            pltpu.SemaphoreType.DMA,
        ],
    )(_scatter_rows_body)


def _combine_rows_body(ys_hbm, p0_hbm, p1_hbm, g0_hbm, g1_hbm, out_hbm,
                       i0_v, i1_v, g0_v, g1_v, r0_v, r1_v, s0, s1):
    wid = lax.axis_index("s") * 2 + lax.axis_index("c")
    base = wid * TPW

    def chunk(c, carry):
        off = base + c * CCH
        pltpu.sync_copy(p0_hbm.at[pl.ds(off, CCH)], i0_v)
        pltpu.sync_copy(p1_hbm.at[pl.ds(off, CCH)], i1_v)
        cp0 = pltpu.async_copy(ys_hbm.at[i0_v], r0_v, s0)
        cp1 = pltpu.async_copy(ys_hbm.at[i1_v], r1_v, s1)
        pltpu.sync_copy(g0_hbm.at[pl.ds(off, CCH)], g0_v)
        pltpu.sync_copy(g1_hbm.at[pl.ds(off, CCH)], g1_v)
        cp0.wait()
        cp1.wait()

        def row(i, rc):
            gv0 = g0_v[i, :]
            gv1 = g1_v[i, :]
            for j in range(D_MODEL // 16):
                sl = pl.ds(j * 16, 16)
                r0_v[i, sl] = gv0 * r0_v[i, sl] + gv1 * r1_v[i, sl]
            return rc

        lax.fori_loop(0, CCH, row, 0)
        pltpu.sync_copy(r0_v, out_hbm.at[pl.ds(off, CCH)])
        return carry

    lax.fori_loop(0, TPW // CCH, chunk, 0)


@functools.cache
def _combine_rows():
    return functools.partial(
        pl.kernel, mesh=_sc_mesh(),
        out_type=jax.ShapeDtypeStruct((T, D_MODEL), jnp.float32),
        scratch_types=[
            pltpu.VMEM((CCH,), jnp.int32),
            pltpu.VMEM((CCH,), jnp.int32),
            pltpu.VMEM((CCH, 16), jnp.float32),
            pltpu.VMEM((CCH, 16), jnp.float32),
            pltpu.VMEM((CCH, D_MODEL), jnp.float32),
            pltpu.VMEM((CCH, D_MODEL), jnp.float32),
            pltpu.SemaphoreType.DMA,
            pltpu.SemaphoreType.DMA,
        ],
    )(_combine_rows_body)


def kernel(x, Wg, W1, W3, W2):
    b, s, d = x.shape
    xf = x.reshape(-1, d)

    p0m, p1m, g0b, g1b, tem = _router(xf, Wg)
    p0 = p0m.reshape(T)
    p1 = p1m.reshape(T)
    te = tem[0, :N_TILES]
    nl = tem[0, 64:65]

    xs = _scatter_rows()(xf, p0, p1)
    ys = _ffn(te, nl, xs, W1, W3, W2)
    outf = _combine_rows()(ys, p0, p1, g0b, g1b)

    return outf.reshape(b, s, d), jnp.asarray(0.0, x.dtype)
